# Initial kernel scaffold; baseline (speedup 1.0000x reference)
import jax
import jax.numpy as jnp
from jax.experimental import pallas as pl


def kernel(x, resolution, origin):
    raise NotImplementedError("write your pallas kernel here")


# speedup vs baseline: 7.6028x; 7.6028x over previous
import functools

import jax
import jax.numpy as jnp
from jax import lax
from jax.experimental import pallas as pl
from jax.experimental.pallas import tpu as pltpu
from jax.experimental.pallas import tpu_sc as plsc

_B = 16
_T = 50
_NP = 3
_H = 100
_W = 100
_NS = 200
_SLICES = _B * _T
_NW = 32
_PER_W = _SLICES // _NW
_PIX = _H * _W * 3
_CH = 13
_MAGIC = 12582912.0


def _body(segp_h, midx_h, mval_h, tpad_h, zeros_h, out_h,
          segp_v, midx_v, mval_v, tpad_v, img_v, idxs_v):
    c = lax.axis_index("c")
    s = lax.axis_index("s")
    w = s * 2 + c
    base = w * _PER_W

    pltpu.sync_copy(segp_h.at[pl.ds(base * 128, _PER_W * 128)], segp_v)
    pltpu.sync_copy(midx_h.at[pl.ds(base * 16, _PER_W * 16)], midx_v)
    pltpu.sync_copy(mval_h, mval_v)
    pltpu.sync_copy(tpad_h, tpad_v)
    pltpu.sync_copy(zeros_h, img_v)

    lane = lax.iota(jnp.int32, 16)
    mask_red = lane < 6
    mask_green = jnp.logical_and(lane >= 6, lane < 9)
    mask_mark = lane < 9
    v128 = jnp.full((16,), 128.0, jnp.float32)
    zf = jnp.zeros((16,), jnp.float32)
    mval = mval_v[...]

    def slice_body(i, carry):
        for seg in range(2):
            r0 = segp_v[pl.ds(i * 128 + (4 * seg + 0) * 16, 16)]
            dr = segp_v[pl.ds(i * 128 + (4 * seg + 1) * 16, 16)]
            c0 = segp_v[pl.ds(i * 128 + (4 * seg + 2) * 16, 16)]
            dc = segp_v[pl.ds(i * 128 + (4 * seg + 3) * 16, 16)]
            for ch in range(_CH):
                tv = tpad_v[ch]
                rf = r0 + tv * dr
                cf = c0 + tv * dc
                rr = jnp.minimum(jnp.maximum((rf + _MAGIC) - _MAGIC,
                                             jnp.float32(0.0)),
                                 jnp.float32(99.0))
                cc = jnp.minimum(jnp.maximum((cf + _MAGIC) - _MAGIC,
                                             jnp.float32(0.0)),
                                 jnp.float32(99.0))
                ri = rr.astype(jnp.int32)
                ci = cc.astype(jnp.int32)
                idx = (ri * 100 + ci) * 3
                plsc.store_scatter(img_v, [idx], v128)
                plsc.store_scatter(img_v, [idx + 1], v128)
                plsc.store_scatter(img_v, [idx + 2], v128)
                idxs_v[pl.ds((seg * _CH + ch) * 16, 16)] = idx
        mi = midx_v[pl.ds(i * 16, 16)]
        plsc.store_scatter(img_v, [mi], mval, mask=mask_red)
        plsc.store_scatter(img_v, [mi], mval, mask=mask_green)
        idxs_v[pl.ds(2 * _CH * 16, 16)] = mi

        pltpu.sync_copy(img_v, out_h.at[pl.ds((base + i) * _PIX, _PIX)])

        for ch in range(2 * _CH):
            idx = idxs_v[pl.ds(ch * 16, 16)]
            plsc.store_scatter(img_v, [idx], zf)
            plsc.store_scatter(img_v, [idx + 1], zf)
            plsc.store_scatter(img_v, [idx + 2], zf)
        mi2 = idxs_v[pl.ds(2 * _CH * 16, 16)]
        plsc.store_scatter(img_v, [mi2], zf, mask=mask_mark)
        return carry

    lax.fori_loop(0, _PER_W, slice_body, 0)


@jax.jit
def _draw(segp, midx, mval, tpad, zeros_row):
    mesh = plsc.VectorSubcoreMesh(core_axis_name="c", subcore_axis_name="s")
    f = pl.kernel(
        _body,
        mesh=mesh,
        compiler_params=pltpu.CompilerParams(needs_layout_passes=False),
        out_type=jax.ShapeDtypeStruct((_SLICES * _PIX,), jnp.float32),
        scratch_types=[
            pltpu.VMEM((_PER_W * 128,), jnp.float32),
            pltpu.VMEM((_PER_W * 16,), jnp.int32),
            pltpu.VMEM((16,), jnp.float32),
            pltpu.VMEM((_CH, 16), jnp.float32),
            pltpu.VMEM((_PIX,), jnp.float32),
            pltpu.VMEM(((2 * _CH + 1) * 16,), jnp.int32),
        ],
    )
    return f(segp, midx, mval, tpad, zeros_row)


def kernel(x, resolution, origin):
    pts = jnp.reshape(x, (_SLICES, _NP, 2))
    px = pts[..., 0]
    py = pts[..., 1]
    rows_f = jnp.clip(jnp.round(py / resolution[0] + origin[0]), 0, _H - 1)
    cols_f = jnp.clip(jnp.round(px / resolution[1] + origin[1]), 0, _W - 1)

    r0 = rows_f[:, :2]
    dr = rows_f[:, 1:] - rows_f[:, :2]
    c0 = cols_f[:, :2]
    dc = cols_f[:, 1:] - cols_f[:, :2]
    segp = jnp.stack(
        [r0[:, 0], dr[:, 0], c0[:, 0], dc[:, 0],
         r0[:, 1], dr[:, 1], c0[:, 1], dc[:, 1]], axis=1)
    segp = jnp.broadcast_to(segp[:, :, None],
                            (_SLICES, 8, 16)).reshape(_SLICES * 128)

    ri = rows_f.astype(jnp.int32)
    ci = cols_f.astype(jnp.int32)
    pix3 = (ri * _W + ci) * 3
    g = jnp.array([0, 0, 0, 1, 1, 1, 2, 2, 2], jnp.int32)
    chan = jnp.array([0, 1, 2, 0, 1, 2, 0, 1, 2], jnp.int32)
    midx = pix3[:, g] + chan[None, :]
    midx = jnp.concatenate(
        [midx, jnp.zeros((_SLICES, 7), jnp.int32)], axis=1)
    midx = midx.reshape(_SLICES * 16)

    mval = jnp.array([255., 0., 0., 255., 0., 0., 0., 255., 0.,
                      0., 0., 0., 0., 0., 0., 0.], jnp.float32)

    t = jnp.linspace(0.0, 1.0, _NS)
    tpad = jnp.concatenate([t, jnp.full((_CH * 16 - _NS,), t[-1],
                                        jnp.float32)]).reshape(_CH, 16)

    zeros_row = jnp.zeros((_PIX,), jnp.float32)

    out = _draw(segp, midx, mval, tpad, zeros_row)
    return jnp.reshape(out, (_B, _T, _H, _W, 3))


# speedup vs baseline: 218.0145x; 28.6756x over previous
import functools

import jax
import jax.numpy as jnp
from jax import lax
from jax.experimental import pallas as pl
from jax.experimental.pallas import tpu as pltpu
from jax.experimental.pallas import tpu_sc as plsc

_B = 16
_T = 50
_NP = 3
_H = 100
_W = 100
_NS = 200
_SLICES = _B * _T
_NW = 32
_PER_W = _SLICES // _NW
_ROWS = _T * _H * 3 * _B
_CH = 13
_MAGIC = 12582912.0


def _body(segp_h, mrow_h, mcol_h, brow_h, mval_h, tpad_h, zeros_h, out_h,
          segp_v, mrow_v, mcol_v, brow_v, mval_v, tpad_v, img_v, idxs_v,
          idxa_v, idxb_v, sem):
    c = lax.axis_index("c")
    s = lax.axis_index("s")
    w = s * 2 + c
    base = w * _PER_W

    pltpu.sync_copy(segp_h.at[pl.ds(base * 128, _PER_W * 128)], segp_v)
    pltpu.sync_copy(mrow_h.at[pl.ds(base * 16, _PER_W * 16)], mrow_v)
    pltpu.sync_copy(mcol_h.at[pl.ds(base * 16, _PER_W * 16)], mcol_v)
    pltpu.sync_copy(brow_h.at[pl.ds(base * 16, _PER_W * 16)], brow_v)
    pltpu.sync_copy(mval_h, mval_v)
    pltpu.sync_copy(tpad_h, tpad_v)
    pltpu.sync_copy(zeros_h, img_v)

    lane = lax.iota(jnp.int32, 16)
    lane48 = lane * 48
    mask_red = lane < 6
    mask_green = jnp.logical_and(lane >= 6, lane < 9)
    mask_mark = lane < 9
    v128 = jnp.full((16,), 128.0, jnp.float32)
    zf = jnp.zeros((16,), jnp.float32)
    mval = mval_v[...]

    def slice_body(i, carry):
        for seg in range(2):
            r0 = segp_v[pl.ds(i * 128 + (4 * seg + 0) * 16, 16)]
            dr = segp_v[pl.ds(i * 128 + (4 * seg + 1) * 16, 16)]
            c0 = segp_v[pl.ds(i * 128 + (4 * seg + 2) * 16, 16)]
            dc = segp_v[pl.ds(i * 128 + (4 * seg + 3) * 16, 16)]
            for ch in range(_CH):
                tv = tpad_v[ch]
                rf = r0 + tv * dr
                cf = c0 + tv * dc
                rr = jnp.minimum(jnp.maximum((rf + _MAGIC) - _MAGIC,
                                             jnp.float32(0.0)),
                                 jnp.float32(99.0))
                cc = jnp.minimum(jnp.maximum((cf + _MAGIC) - _MAGIC,
                                             jnp.float32(0.0)),
                                 jnp.float32(99.0))
                ri = rr.astype(jnp.int32)
                wi = cc.astype(jnp.int32)
                plsc.store_scatter(img_v, [ri, wi], v128)
                plsc.store_scatter(img_v, [ri + 100, wi], v128)
                plsc.store_scatter(img_v, [ri + 200, wi], v128)
                k = (seg * _CH + ch) * 32
                idxs_v[pl.ds(k, 16)] = ri
                idxs_v[pl.ds(k + 16, 16)] = wi
        mr = mrow_v[pl.ds(i * 16, 16)]
        mc = mcol_v[pl.ds(i * 16, 16)]
        plsc.store_scatter(img_v, [mr, mc], mval, mask=mask_red)
        plsc.store_scatter(img_v, [mr, mc], mval, mask=mask_green)

        brow = brow_v[pl.ds(i * 16, 16)]
        hs = []
        for chn in range(3):
            for j in range(6):
                idxa_v[chn, pl.ds(16 * j, 16)] = brow + (lane48 + (768 * j + 16 * chn))
            idxb_v[chn, pl.ds(0, 16)] = brow + (lane48 + (4032 + 16 * chn))
            hs.append(pltpu.async_copy(
                img_v.at[pl.ds(100 * chn, 96)], out_h.at[idxa_v.at[chn]], sem))
            hs.append(pltpu.async_copy(
                img_v.at[pl.ds(100 * chn + 84, 16)], out_h.at[idxb_v.at[chn]], sem))
        for h in hs:
            h.wait()

        for ch in range(2 * _CH):
            ri = idxs_v[pl.ds(ch * 32, 16)]
            wi = idxs_v[pl.ds(ch * 32 + 16, 16)]
            plsc.store_scatter(img_v, [ri, wi], zf)
            plsc.store_scatter(img_v, [ri + 100, wi], zf)
            plsc.store_scatter(img_v, [ri + 200, wi], zf)
        plsc.store_scatter(img_v, [mr, mc], zf, mask=mask_mark)
        return carry

    lax.fori_loop(0, _PER_W, slice_body, 0)


@jax.jit
def _draw(segp, mrow, mcol, brow, mval, tpad, zeros_img):
    mesh = plsc.VectorSubcoreMesh(core_axis_name="c", subcore_axis_name="s")
    f = pl.kernel(
        _body,
        mesh=mesh,
        compiler_params=pltpu.CompilerParams(needs_layout_passes=False),
        out_type=jax.ShapeDtypeStruct((_ROWS, 128), jnp.float32),
        scratch_types=[
            pltpu.VMEM((_PER_W * 128,), jnp.float32),
            pltpu.VMEM((_PER_W * 16,), jnp.int32),
            pltpu.VMEM((_PER_W * 16,), jnp.int32),
            pltpu.VMEM((_PER_W * 16,), jnp.int32),
            pltpu.VMEM((16,), jnp.float32),
            pltpu.VMEM((_CH, 16), jnp.float32),
            pltpu.VMEM((300, 128), jnp.float32),
            pltpu.VMEM((2 * _CH * 32,), jnp.int32),
            pltpu.VMEM((3, 96), jnp.int32),
            pltpu.VMEM((3, 16), jnp.int32),
            pltpu.SemaphoreType.DMA,
        ],
    )
    return f(segp, mrow, mcol, brow, mval, tpad, zeros_img)


def kernel(x, resolution, origin):
    pts = jnp.reshape(x, (_SLICES, _NP, 2))
    px = pts[..., 0]
    py = pts[..., 1]
    rows_f = jnp.clip(jnp.round(py / resolution[0] + origin[0]), 0, _H - 1)
    cols_f = jnp.clip(jnp.round(px / resolution[1] + origin[1]), 0, _W - 1)

    r0 = rows_f[:, :2]
    dr = rows_f[:, 1:] - rows_f[:, :2]
    c0 = cols_f[:, :2]
    dc = cols_f[:, 1:] - cols_f[:, :2]
    segp = jnp.stack(
        [r0[:, 0], dr[:, 0], c0[:, 0], dc[:, 0],
         r0[:, 1], dr[:, 1], c0[:, 1], dc[:, 1]], axis=1)
    segp = jnp.broadcast_to(segp[:, :, None],
                            (_SLICES, 8, 16)).reshape(_SLICES * 128)

    ri = rows_f.astype(jnp.int32)
    ci = cols_f.astype(jnp.int32)
    g = jnp.array([0, 0, 0, 1, 1, 1, 2, 2, 2], jnp.int32)
    chan = jnp.array([0, 1, 2, 0, 1, 2, 0, 1, 2], jnp.int32)
    mrow = ri[:, g] + chan[None, :] * 100
    mcol = ci[:, g]
    pad = jnp.zeros((_SLICES, 7), jnp.int32)
    mrow = jnp.concatenate([mrow, pad], axis=1).reshape(_SLICES * 16)
    mcol = jnp.concatenate([mcol, pad], axis=1).reshape(_SLICES * 16)

    sidx = jnp.arange(_SLICES, dtype=jnp.int32)
    tt = sidx % _T
    bb = sidx // _T
    brow = jnp.broadcast_to((tt * (_H * 3 * _B) + bb)[:, None],
                            (_SLICES, 16)).reshape(_SLICES * 16)

    mval = jnp.array([255., 0., 0., 255., 0., 0., 0., 255., 0.,
                      0., 0., 0., 0., 0., 0., 0.], jnp.float32)

    t = jnp.linspace(0.0, 1.0, _NS)
    tpad = jnp.concatenate([t, jnp.full((_CH * 16 - _NS,), t[-1],
                                        jnp.float32)]).reshape(_CH, 16)

    zeros_img = jnp.zeros((300, 128), jnp.float32)

    out2 = _draw(segp, mrow, mcol, brow, mval, tpad, zeros_img)
    out5 = jnp.reshape(out2, (_T, _H, 3, _B, 128))[..., :_W]
    return jnp.transpose(out5, (3, 0, 1, 4, 2))


# speedup vs baseline: 223.3135x; 1.0243x over previous
import functools

import jax
import jax.numpy as jnp
from jax import lax
from jax.experimental import pallas as pl
from jax.experimental.pallas import tpu as pltpu
from jax.experimental.pallas import tpu_sc as plsc

_B = 16
_T = 50
_NP = 3
_H = 100
_W = 100
_NS = 200
_SLICES = _B * _T
_NW = 32
_PER_W = _SLICES // _NW
_ROWS = _T * _H * 3 * _B
_CH = 13
_MAGIC = 12582912.0


def _body(segp_h, mrow_h, mcol_h, brow_h, mval_h, tpad_h, zeros_h, out_h,
          segp_v, mrow_v, mcol_v, brow_v, mval_v, tpad_v, img_v, idxs_v,
          idxa_v, idxb_v, sem):
    c = lax.axis_index("c")
    s = lax.axis_index("s")
    w = s * 2 + c
    base = w * _PER_W

    pltpu.sync_copy(segp_h.at[pl.ds(base * 128, _PER_W * 128)], segp_v)
    pltpu.sync_copy(mrow_h.at[pl.ds(base * 16, _PER_W * 16)], mrow_v)
    pltpu.sync_copy(mcol_h.at[pl.ds(base * 16, _PER_W * 16)], mcol_v)
    pltpu.sync_copy(brow_h.at[pl.ds(base * 16, _PER_W * 16)], brow_v)
    pltpu.sync_copy(mval_h, mval_v)
    pltpu.sync_copy(tpad_h, tpad_v)
    pltpu.sync_copy(zeros_h, img_v)

    lane = lax.iota(jnp.int32, 16)
    lane48 = lane * 48
    mask_red = lane < 6
    mask_green = jnp.logical_and(lane >= 6, lane < 9)
    mask_mark = lane < 9
    v128 = jnp.full((16,), 128.0, jnp.float32)
    zf = jnp.zeros((16,), jnp.float32)
    mval = mval_v[...]

    def slice_body(i, carry):
        for seg in range(2):
            r0 = segp_v[pl.ds(i * 128 + (4 * seg + 0) * 16, 16)]
            dr = segp_v[pl.ds(i * 128 + (4 * seg + 1) * 16, 16)]
            c0 = segp_v[pl.ds(i * 128 + (4 * seg + 2) * 16, 16)]
            dc = segp_v[pl.ds(i * 128 + (4 * seg + 3) * 16, 16)]
            for ch in range(_CH):
                tv = tpad_v[ch]
                rf = r0 + tv * dr
                cf = c0 + tv * dc
                rr = jnp.minimum(jnp.maximum((rf + _MAGIC) - _MAGIC,
                                             jnp.float32(0.0)),
                                 jnp.float32(99.0))
                cc = jnp.minimum(jnp.maximum((cf + _MAGIC) - _MAGIC,
                                             jnp.float32(0.0)),
                                 jnp.float32(99.0))
                ri = rr.astype(jnp.int32)
                wi = cc.astype(jnp.int32)
                plsc.store_scatter(img_v, [ri, wi], v128)
                plsc.store_scatter(img_v, [ri + 100, wi], v128)
                plsc.store_scatter(img_v, [ri + 200, wi], v128)
                k = (seg * _CH + ch) * 32
                idxs_v[pl.ds(k, 16)] = ri
                idxs_v[pl.ds(k + 16, 16)] = wi
        mr = mrow_v[pl.ds(i * 16, 16)]
        mc = mcol_v[pl.ds(i * 16, 16)]
        plsc.store_scatter(img_v, [mr, mc], mval, mask=mask_red)
        plsc.store_scatter(img_v, [mr, mc], mval, mask=mask_green)

        brow = brow_v[pl.ds(i * 16, 16)]
        hs = []
        for chn in range(3):
            for j in range(6):
                idxa_v[chn, pl.ds(16 * j, 16)] = brow + (lane48 + (768 * j + 16 * chn))
            idxa_v[chn, pl.ds(84, 16)] = brow + (lane48 + (4032 + 16 * chn))
            hs.append(pltpu.async_copy(
                img_v.at[pl.ds(100 * chn, 100)], out_h.at[idxa_v.at[chn]], sem))
        for h in hs:
            h.wait()

        for ch in range(2 * _CH):
            ri = idxs_v[pl.ds(ch * 32, 16)]
            wi = idxs_v[pl.ds(ch * 32 + 16, 16)]
            plsc.store_scatter(img_v, [ri, wi], zf)
            plsc.store_scatter(img_v, [ri + 100, wi], zf)
            plsc.store_scatter(img_v, [ri + 200, wi], zf)
        plsc.store_scatter(img_v, [mr, mc], zf, mask=mask_mark)
        return carry

    lax.fori_loop(0, _PER_W, slice_body, 0)


@jax.jit
def _draw(segp, mrow, mcol, brow, mval, tpad, zeros_img):
    mesh = plsc.VectorSubcoreMesh(core_axis_name="c", subcore_axis_name="s")
    f = pl.kernel(
        _body,
        mesh=mesh,
        compiler_params=pltpu.CompilerParams(needs_layout_passes=False),
        out_type=jax.ShapeDtypeStruct((_ROWS, 128), jnp.float32),
        scratch_types=[
            pltpu.VMEM((_PER_W * 128,), jnp.float32),
            pltpu.VMEM((_PER_W * 16,), jnp.int32),
            pltpu.VMEM((_PER_W * 16,), jnp.int32),
            pltpu.VMEM((_PER_W * 16,), jnp.int32),
            pltpu.VMEM((16,), jnp.float32),
            pltpu.VMEM((_CH, 16), jnp.float32),
            pltpu.VMEM((300, 128), jnp.float32),
            pltpu.VMEM((2 * _CH * 32,), jnp.int32),
            pltpu.VMEM((3, 100), jnp.int32),
            pltpu.VMEM((3, 16), jnp.int32),
            pltpu.SemaphoreType.DMA,
        ],
    )
    return f(segp, mrow, mcol, brow, mval, tpad, zeros_img)


def kernel(x, resolution, origin):
    pts = jnp.reshape(x, (_SLICES, _NP, 2))
    px = pts[..., 0]
    py = pts[..., 1]
    rows_f = jnp.clip(jnp.round(py / resolution[0] + origin[0]), 0, _H - 1)
    cols_f = jnp.clip(jnp.round(px / resolution[1] + origin[1]), 0, _W - 1)

    r0 = rows_f[:, :2]
    dr = rows_f[:, 1:] - rows_f[:, :2]
    c0 = cols_f[:, :2]
    dc = cols_f[:, 1:] - cols_f[:, :2]
    segp = jnp.stack(
        [r0[:, 0], dr[:, 0], c0[:, 0], dc[:, 0],
         r0[:, 1], dr[:, 1], c0[:, 1], dc[:, 1]], axis=1)
    segp = jnp.broadcast_to(segp[:, :, None],
                            (_SLICES, 8, 16)).reshape(_SLICES * 128)

    ri = rows_f.astype(jnp.int32)
    ci = cols_f.astype(jnp.int32)
    g = jnp.array([0, 0, 0, 1, 1, 1, 2, 2, 2], jnp.int32)
    chan = jnp.array([0, 1, 2, 0, 1, 2, 0, 1, 2], jnp.int32)
    mrow = ri[:, g] + chan[None, :] * 100
    mcol = ci[:, g]
    pad = jnp.zeros((_SLICES, 7), jnp.int32)
    mrow = jnp.concatenate([mrow, pad], axis=1).reshape(_SLICES * 16)
    mcol = jnp.concatenate([mcol, pad], axis=1).reshape(_SLICES * 16)

    sidx = jnp.arange(_SLICES, dtype=jnp.int32)
    tt = sidx % _T
    bb = sidx // _T
    brow = jnp.broadcast_to((tt * (_H * 3 * _B) + bb)[:, None],
                            (_SLICES, 16)).reshape(_SLICES * 16)

    mval = jnp.array([255., 0., 0., 255., 0., 0., 0., 255., 0.,
                      0., 0., 0., 0., 0., 0., 0.], jnp.float32)

    t = jnp.linspace(0.0, 1.0, _NS)
    tpad = jnp.concatenate([t, jnp.full((_CH * 16 - _NS,), t[-1],
                                        jnp.float32)]).reshape(_CH, 16)

    zeros_img = jnp.zeros((300, 128), jnp.float32)

    out2 = _draw(segp, mrow, mcol, brow, mval, tpad, zeros_img)
    out5 = jnp.reshape(out2, (_T, _H, 3, _B, 128))[..., :_W]
    return jnp.transpose(out5, (3, 0, 1, 4, 2))


# speedup vs baseline: 238.2854x; 1.0670x over previous
import functools

import jax
import jax.numpy as jnp
from jax import lax
from jax.experimental import pallas as pl
from jax.experimental.pallas import tpu as pltpu
from jax.experimental.pallas import tpu_sc as plsc

_B = 16
_T = 50
_NP = 3
_H = 100
_W = 100
_NS = 200
_SLICES = _B * _T
_NW = 32
_PER_W = _SLICES // _NW
_ROWS = _T * _H * 3 * _B
_CH = 13
_MAGIC = 12582912.0


def _body(segp_h, mrow_h, mcol_h, brow_h, mval_h, tpad_h, zeros_h, out_h,
          segp_v, mrow_v, mcol_v, brow_v, mval_v, tpad_v,
          img0_v, img1_v, idxs0_v, idxs1_v, idxa0_v, idxa1_v, sem0, sem1):
    c = lax.axis_index("c")
    s = lax.axis_index("s")
    w = s * 2 + c
    base = w * _PER_W

    pltpu.sync_copy(segp_h.at[pl.ds(base * 128, _PER_W * 128)], segp_v)
    pltpu.sync_copy(mrow_h.at[pl.ds(base * 16, _PER_W * 16)], mrow_v)
    pltpu.sync_copy(mcol_h.at[pl.ds(base * 16, _PER_W * 16)], mcol_v)
    pltpu.sync_copy(brow_h.at[pl.ds(base * 16, _PER_W * 16)], brow_v)
    pltpu.sync_copy(mval_h, mval_v)
    pltpu.sync_copy(tpad_h, tpad_v)
    pltpu.sync_copy(zeros_h, img0_v)
    pltpu.sync_copy(zeros_h, img1_v)

    lane = lax.iota(jnp.int32, 16)
    lane48 = lane * 48
    mask_red = lane < 6
    mask_green = jnp.logical_and(lane >= 6, lane < 9)
    mask_mark = lane < 9
    v128 = jnp.full((16,), 128.0, jnp.float32)
    zf = jnp.zeros((16,), jnp.float32)
    mval = mval_v[...]

    def wait_bufs(img_v, idxa_v, sem):
        for chn in range(3):
            pltpu.make_async_copy(
                img_v.at[pl.ds(100 * chn, 100)],
                out_h.at[idxa_v.at[chn]], sem).wait()

    def cleanup(img_v, idxs_v, mr, mc):
        for ch in range(2 * _CH):
            ri = idxs_v[pl.ds(ch * 32, 16)]
            wi = idxs_v[pl.ds(ch * 32 + 16, 16)]
            plsc.store_scatter(img_v, [ri, wi], zf)
            plsc.store_scatter(img_v, [ri + 100, wi], zf)
            plsc.store_scatter(img_v, [ri + 200, wi], zf)
        plsc.store_scatter(img_v, [mr, mc], zf, mask=mask_mark)

    def do_slice(i, img_v, idxs_v, idxa_v, sem, first):
        if not first:
            wait_bufs(img_v, idxa_v, sem)
            mr_p = mrow_v[pl.ds((i - 2) * 16, 16)]
            mc_p = mcol_v[pl.ds((i - 2) * 16, 16)]
            cleanup(img_v, idxs_v, mr_p, mc_p)

        for seg in range(2):
            r0 = segp_v[pl.ds(i * 128 + (4 * seg + 0) * 16, 16)]
            dr = segp_v[pl.ds(i * 128 + (4 * seg + 1) * 16, 16)]
            c0 = segp_v[pl.ds(i * 128 + (4 * seg + 2) * 16, 16)]
            dc = segp_v[pl.ds(i * 128 + (4 * seg + 3) * 16, 16)]
            for ch in range(_CH):
                tv = tpad_v[ch]
                rf = r0 + tv * dr
                cf = c0 + tv * dc
                rr = jnp.minimum(jnp.maximum((rf + _MAGIC) - _MAGIC,
                                             jnp.float32(0.0)),
                                 jnp.float32(99.0))
                cc = jnp.minimum(jnp.maximum((cf + _MAGIC) - _MAGIC,
                                             jnp.float32(0.0)),
                                 jnp.float32(99.0))
                ri = rr.astype(jnp.int32)
                wi = cc.astype(jnp.int32)
                plsc.store_scatter(img_v, [ri, wi], v128)
                plsc.store_scatter(img_v, [ri + 100, wi], v128)
                plsc.store_scatter(img_v, [ri + 200, wi], v128)
                k = (seg * _CH + ch) * 32
                idxs_v[pl.ds(k, 16)] = ri
                idxs_v[pl.ds(k + 16, 16)] = wi
        mr = mrow_v[pl.ds(i * 16, 16)]
        mc = mcol_v[pl.ds(i * 16, 16)]
        plsc.store_scatter(img_v, [mr, mc], mval, mask=mask_red)
        plsc.store_scatter(img_v, [mr, mc], mval, mask=mask_green)

        brow = brow_v[pl.ds(i * 16, 16)]
        for chn in range(3):
            for j in range(6):
                idxa_v[chn, pl.ds(16 * j, 16)] = brow + (lane48 + (768 * j + 16 * chn))
            idxa_v[chn, pl.ds(84, 16)] = brow + (lane48 + (4032 + 16 * chn))
            pltpu.async_copy(
                img_v.at[pl.ds(100 * chn, 100)], out_h.at[idxa_v.at[chn]], sem)

    def pair_body(j, carry):
        do_slice(2 * j, img0_v, idxs0_v, idxa0_v, sem0, False)
        do_slice(2 * j + 1, img1_v, idxs1_v, idxa1_v, sem1, False)
        return carry

    do_slice(0, img0_v, idxs0_v, idxa0_v, sem0, True)
    do_slice(1, img1_v, idxs1_v, idxa1_v, sem1, True)
    lax.fori_loop(1, 12, pair_body, 0)
    do_slice(24, img0_v, idxs0_v, idxa0_v, sem0, False)
    wait_bufs(img0_v, idxa0_v, sem0)
    wait_bufs(img1_v, idxa1_v, sem1)


@jax.jit
def _draw(segp, mrow, mcol, brow, mval, tpad, zeros_img):
    mesh = plsc.VectorSubcoreMesh(core_axis_name="c", subcore_axis_name="s")
    f = pl.kernel(
        _body,
        mesh=mesh,
        compiler_params=pltpu.CompilerParams(needs_layout_passes=False),
        out_type=jax.ShapeDtypeStruct((_ROWS, 128), jnp.float32),
        scratch_types=[
            pltpu.VMEM((_PER_W * 128,), jnp.float32),
            pltpu.VMEM((_PER_W * 16,), jnp.int32),
            pltpu.VMEM((_PER_W * 16,), jnp.int32),
            pltpu.VMEM((_PER_W * 16,), jnp.int32),
            pltpu.VMEM((16,), jnp.float32),
            pltpu.VMEM((_CH, 16), jnp.float32),
            pltpu.VMEM((300, 128), jnp.float32),
            pltpu.VMEM((300, 128), jnp.float32),
            pltpu.VMEM((2 * _CH * 32,), jnp.int32),
            pltpu.VMEM((2 * _CH * 32,), jnp.int32),
            pltpu.VMEM((3, 100), jnp.int32),
            pltpu.VMEM((3, 100), jnp.int32),
            pltpu.SemaphoreType.DMA,
            pltpu.SemaphoreType.DMA,
        ],
    )
    return f(segp, mrow, mcol, brow, mval, tpad, zeros_img)


def kernel(x, resolution, origin):
    pts = jnp.reshape(x, (_SLICES, _NP, 2))
    px = pts[..., 0]
    py = pts[..., 1]
    rows_f = jnp.clip(jnp.round(py / resolution[0] + origin[0]), 0, _H - 1)
    cols_f = jnp.clip(jnp.round(px / resolution[1] + origin[1]), 0, _W - 1)

    r0 = rows_f[:, :2]
    dr = rows_f[:, 1:] - rows_f[:, :2]
    c0 = cols_f[:, :2]
    dc = cols_f[:, 1:] - cols_f[:, :2]
    segp = jnp.stack(
        [r0[:, 0], dr[:, 0], c0[:, 0], dc[:, 0],
         r0[:, 1], dr[:, 1], c0[:, 1], dc[:, 1]], axis=1)
    segp = jnp.broadcast_to(segp[:, :, None],
                            (_SLICES, 8, 16)).reshape(_SLICES * 128)

    ri = rows_f.astype(jnp.int32)
    ci = cols_f.astype(jnp.int32)
    chan = jnp.arange(3, dtype=jnp.int32)
    mrow = (jnp.broadcast_to(ri[:, :, None], (_SLICES, 3, 3))
            + chan[None, None, :] * 100).reshape(_SLICES, 9)
    mcol = jnp.broadcast_to(ci[:, :, None], (_SLICES, 3, 3)).reshape(_SLICES, 9)
    pad = jnp.zeros((_SLICES, 7), jnp.int32)
    mrow = jnp.concatenate([mrow, pad], axis=1).reshape(_SLICES * 16)
    mcol = jnp.concatenate([mcol, pad], axis=1).reshape(_SLICES * 16)

    sidx = jnp.arange(_SLICES, dtype=jnp.int32)
    tt = sidx % _T
    bb = sidx // _T
    brow = jnp.broadcast_to((tt * (_H * 3 * _B) + bb)[:, None],
                            (_SLICES, 16)).reshape(_SLICES * 16)

    mval = jnp.array([255., 0., 0., 255., 0., 0., 0., 255., 0.,
                      0., 0., 0., 0., 0., 0., 0.], jnp.float32)

    t = jnp.linspace(0.0, 1.0, _NS)
    tpad = jnp.concatenate([t, jnp.full((_CH * 16 - _NS,), t[-1],
                                        jnp.float32)]).reshape(_CH, 16)

    zeros_img = jnp.zeros((300, 128), jnp.float32)

    out2 = _draw(segp, mrow, mcol, brow, mval, tpad, zeros_img)
    out5 = jnp.reshape(out2, (_T, _H, 3, _B, 128))[..., :_W]
    return jnp.transpose(out5, (3, 0, 1, 4, 2))


# speedup vs baseline: 238.4200x; 1.0006x over previous
import jax
import jax.numpy as jnp
from jax import lax
from jax.experimental import pallas as pl
from jax.experimental.pallas import tpu as pltpu
from jax.experimental.pallas import tpu_sc as plsc

_B = 16
_T = 50
_NP = 3
_H = 100
_W = 100
_NS = 200
_SLICES = _B * _T
_NW = 32
_PER_W = _SLICES // _NW
_ROWS = _T * _H * 3 * _B
_CH = 13
_MAGIC = 12582912.0


def _body(segp_h, mrow_h, mcol_h, brow_h, mval_h, tpad_h, zeros_h, out_h,
          segp_v, mrow_v, mcol_v, brow_v, mval_v, tpad_v,
          img0_v, img1_v, idxs0_v, idxs1_v, idxa0_v, idxa1_v, sem0, sem1):
    c = lax.axis_index("c")
    s = lax.axis_index("s")
    w = s * 2 + c
    base = w * _PER_W

    pltpu.sync_copy(segp_h.at[pl.ds(base * 128, _PER_W * 128)], segp_v)
    pltpu.sync_copy(mrow_h.at[pl.ds(base * 16, _PER_W * 16)], mrow_v)
    pltpu.sync_copy(mcol_h.at[pl.ds(base * 16, _PER_W * 16)], mcol_v)
    pltpu.sync_copy(brow_h.at[pl.ds(base * 16, _PER_W * 16)], brow_v)
    pltpu.sync_copy(mval_h, mval_v)
    pltpu.sync_copy(tpad_h, tpad_v)
    pltpu.sync_copy(zeros_h, img0_v)
    pltpu.sync_copy(zeros_h, img1_v)

    lane = lax.iota(jnp.int32, 16)
    lane48 = lane * 48
    mask_red = lane < 6
    mask_green = jnp.logical_and(lane >= 6, lane < 9)
    mask_mark = lane < 9
    v128 = jnp.full((16,), 128.0, jnp.float32)
    zf = jnp.zeros((16,), jnp.float32)
    mval = mval_v[...]

    def wait_bufs(img_v, idxa_v, sem):
        for chn in range(3):
            pltpu.make_async_copy(
                img_v.at[pl.ds(100 * chn, 100)],
                out_h.at[idxa_v.at[chn]], sem).wait()

    def cleanup(img_v, idxs_v, mr, mc):
        for ch in range(2 * _CH):
            ri = idxs_v[pl.ds(ch * 32, 16)]
            wi = idxs_v[pl.ds(ch * 32 + 16, 16)]
            plsc.store_scatter(img_v, [ri, wi], zf)
            plsc.store_scatter(img_v, [ri + 100, wi], zf)
            plsc.store_scatter(img_v, [ri + 200, wi], zf)
        plsc.store_scatter(img_v, [mr, mc], zf, mask=mask_mark)

    def do_slice(i, img_v, idxs_v, idxa_v, sem, first):
        if not first:
            wait_bufs(img_v, idxa_v, sem)
            mr_p = mrow_v[pl.ds((i - 2) * 16, 16)]
            mc_p = mcol_v[pl.ds((i - 2) * 16, 16)]
            cleanup(img_v, idxs_v, mr_p, mc_p)

        for seg in range(2):
            r0 = segp_v[pl.ds(i * 128 + (4 * seg + 0) * 16, 16)]
            dr = segp_v[pl.ds(i * 128 + (4 * seg + 1) * 16, 16)]
            c0 = segp_v[pl.ds(i * 128 + (4 * seg + 2) * 16, 16)]
            dc = segp_v[pl.ds(i * 128 + (4 * seg + 3) * 16, 16)]
            for ch in range(_CH):
                tv = tpad_v[ch]
                rf = r0 + tv * dr
                cf = c0 + tv * dc
                rr = jnp.minimum(jnp.maximum((rf + _MAGIC) - _MAGIC,
                                             jnp.float32(0.0)),
                                 jnp.float32(99.0))
                cc = jnp.minimum(jnp.maximum((cf + _MAGIC) - _MAGIC,
                                             jnp.float32(0.0)),
                                 jnp.float32(99.0))
                ri = rr.astype(jnp.int32)
                wi = cc.astype(jnp.int32)
                plsc.store_scatter(img_v, [ri, wi], v128)
                plsc.store_scatter(img_v, [ri + 100, wi], v128)
                plsc.store_scatter(img_v, [ri + 200, wi], v128)
                k = (seg * _CH + ch) * 32
                idxs_v[pl.ds(k, 16)] = ri
                idxs_v[pl.ds(k + 16, 16)] = wi
        mr = mrow_v[pl.ds(i * 16, 16)]
        mc = mcol_v[pl.ds(i * 16, 16)]
        plsc.store_scatter(img_v, [mr, mc], mval, mask=mask_red)
        plsc.store_scatter(img_v, [mr, mc], mval, mask=mask_green)

        brow = brow_v[pl.ds(i * 16, 16)]
        for chn in range(3):
            for j in range(6):
                idxa_v[chn, pl.ds(16 * j, 16)] = brow + (lane48 + (768 * j + 16 * chn))
            idxa_v[chn, pl.ds(84, 16)] = brow + (lane48 + (4032 + 16 * chn))
            pltpu.async_copy(
                img_v.at[pl.ds(100 * chn, 100)], out_h.at[idxa_v.at[chn]], sem)

    def pair_body(j, carry):
        do_slice(2 * j, img0_v, idxs0_v, idxa0_v, sem0, False)
        do_slice(2 * j + 1, img1_v, idxs1_v, idxa1_v, sem1, False)
        return carry

    do_slice(0, img0_v, idxs0_v, idxa0_v, sem0, True)
    do_slice(1, img1_v, idxs1_v, idxa1_v, sem1, True)
    lax.fori_loop(1, 12, pair_body, 0)
    do_slice(24, img0_v, idxs0_v, idxa0_v, sem0, False)
    wait_bufs(img0_v, idxa0_v, sem0)
    wait_bufs(img1_v, idxa1_v, sem1)


@jax.jit
def _draw(segp, mrow, mcol, brow, mval, tpad, zeros_img):
    mesh = plsc.VectorSubcoreMesh(core_axis_name="c", subcore_axis_name="s")
    f = pl.kernel(
        _body,
        mesh=mesh,
        compiler_params=pltpu.CompilerParams(needs_layout_passes=False),
        out_type=jax.ShapeDtypeStruct((_ROWS, 128), jnp.float32),
        scratch_types=[
            pltpu.VMEM((_PER_W * 128,), jnp.float32),
            pltpu.VMEM((_PER_W * 16,), jnp.int32),
            pltpu.VMEM((_PER_W * 16,), jnp.int32),
            pltpu.VMEM((_PER_W * 16,), jnp.int32),
            pltpu.VMEM((16,), jnp.float32),
            pltpu.VMEM((_CH, 16), jnp.float32),
            pltpu.VMEM((300, 128), jnp.float32),
            pltpu.VMEM((300, 128), jnp.float32),
            pltpu.VMEM((2 * _CH * 32,), jnp.int32),
            pltpu.VMEM((2 * _CH * 32,), jnp.int32),
            pltpu.VMEM((3, 100), jnp.int32),
            pltpu.VMEM((3, 100), jnp.int32),
            pltpu.SemaphoreType.DMA,
            pltpu.SemaphoreType.DMA,
        ],
    )
    return f(segp, mrow, mcol, brow, mval, tpad, zeros_img)


def kernel(x, resolution, origin):
    pts = jnp.reshape(x, (_SLICES, _NP, 2))
    px = pts[..., 0]
    py = pts[..., 1]
    rows_f = jnp.clip(jnp.round(py / resolution[0] + origin[0]), 0, _H - 1)
    cols_f = jnp.clip(jnp.round(px / resolution[1] + origin[1]), 0, _W - 1)

    r0 = rows_f[:, :2]
    dr = rows_f[:, 1:] - rows_f[:, :2]
    c0 = cols_f[:, :2]
    dc = cols_f[:, 1:] - cols_f[:, :2]
    segp = jnp.stack(
        [r0[:, 0], dr[:, 0], c0[:, 0], dc[:, 0],
         r0[:, 1], dr[:, 1], c0[:, 1], dc[:, 1]], axis=1)
    segp = jnp.broadcast_to(segp[:, :, None],
                            (_SLICES, 8, 16)).reshape(_SLICES * 128)

    ri = rows_f.astype(jnp.int32)
    ci = cols_f.astype(jnp.int32)
    chan = jnp.arange(3, dtype=jnp.int32)
    mrow = (jnp.broadcast_to(ri[:, :, None], (_SLICES, 3, 3))
            + chan[None, None, :] * 100).reshape(_SLICES, 9)
    mcol = jnp.broadcast_to(ci[:, :, None], (_SLICES, 3, 3)).reshape(_SLICES, 9)
    pad = jnp.zeros((_SLICES, 7), jnp.int32)
    mrow = jnp.concatenate([mrow, pad], axis=1).reshape(_SLICES * 16)
    mcol = jnp.concatenate([mcol, pad], axis=1).reshape(_SLICES * 16)

    sidx = jnp.arange(_SLICES, dtype=jnp.int32)
    tt = sidx % _T
    bb = sidx // _T
    brow = jnp.broadcast_to((tt * (_H * 3 * _B) + bb)[:, None],
                            (_SLICES, 16)).reshape(_SLICES * 16)

    mval = jnp.array([255., 0., 0., 255., 0., 0., 0., 255., 0.,
                      0., 0., 0., 0., 0., 0., 0.], jnp.float32)

    t = jnp.linspace(0.0, 1.0, _NS)
    tpad = jnp.concatenate([t, jnp.full((_CH * 16 - _NS,), t[-1],
                                        jnp.float32)]).reshape(_CH, 16)

    zeros_img = jnp.zeros((300, 128), jnp.float32)

    out2 = _draw(segp, mrow, mcol, brow, mval, tpad, zeros_img)
    out5 = jnp.reshape(out2, (_T, _H, 3, _B, 128))[..., :_W]
    return jnp.transpose(out5, (3, 0, 1, 4, 2))
